# baseline (speedup 1.0000x reference)
import functools

import jax
import jax.numpy as jnp
from jax import lax
from jax.experimental import pallas as pl
from jax.experimental.pallas import tpu as pltpu
from jax.experimental.pallas import tpu_sc as plsc

_NC = 2
_NS = 16
_L = 16
_NW = _NC * _NS

_D = 8
_V = 16
_B = 16384
_SEQ = 200
_NLT = _SEQ // _D
_NBT = _B // 128
_BT_PER_W = _NBT // _NW
_UNITS = _BT_PER_W * _NLT
_RING = 4

_XTILE = _D * 128
_OTILE = _D * 128
_OUNIT = _D * _OTILE


def _tiny_model_body(x_hbm, emb_hbm, w_hbm, b_hbm, out_hbm,
                     tab_v, x_v, out_v, sem_x, sem_out):
    wid = lax.axis_index("s") * _NC + lax.axis_index("c")
    bt0 = wid * _BT_PER_W

    lanes = lax.iota(jnp.int32, _L)
    o_pat = lanes & 7
    half = lanes >> 3

    def with_weights(emb_v, w_v, b_v):
        pltpu.sync_copy(emb_hbm, emb_v)
        pltpu.sync_copy(w_hbm, w_v)
        pltpu.sync_copy(b_hbm, b_v)

        def build_tab(j, carry):
            k_pat = 2 * j + half
            acc = plsc.load_gather(b_v, [o_pat])
            for d in range(_D):
                ev = plsc.load_gather(emb_v, [k_pat * _D + d])
                wv = plsc.load_gather(w_v, [o_pat * _D + d])
                acc = acc + ev * wv
            tab_v[pl.ds(j * _L, _L)] = acc
            return carry

        lax.fori_loop(0, _V // 2, build_tab, 0)

    pl.run_scoped(
        with_weights,
        pltpu.VMEM((128,), jnp.float32),
        pltpu.VMEM((128,), jnp.float32),
        pltpu.VMEM((128,), jnp.float32),
    )

    def x_off(u):
        lt = lax.rem(u, _NLT)
        bt = bt0 + lax.div(u, _NLT)
        return (lt * _NBT + bt) * _XTILE

    def start_x(u, slot):
        return pltpu.async_copy(
            x_hbm.at[pl.ds(x_off(lax.min(u, _UNITS - 1)), _XTILE)],
            x_v.at[pl.ds(slot * _XTILE, _XTILE)], sem_x[slot])

    def wait_x(u, slot):
        pltpu.make_async_copy(
            x_hbm.at[pl.ds(x_off(lax.min(u, _UNITS - 1)), _XTILE)],
            x_v.at[pl.ds(slot * _XTILE, _XTILE)], sem_x[slot]).wait()

    def drain_out(slot):
        pltpu.make_async_copy(
            out_hbm.at[pl.ds(0, _OUNIT)],
            out_v.at[pl.ds(slot * _OUNIT, _OUNIT)], sem_out[slot]).wait()

    for slot in range(_RING):
        start_x(jnp.int32(slot), slot)

    def k_body(k, carry):
        for p in range(_RING):
            u = k * _RING + p
            lt = lax.rem(u, _NLT)
            bt = bt0 + lax.div(u, _NLT)
            wait_x(u, p)

            @pl.when(k >= 1)
            def _():
                drain_out(p)

            xbase = p * _XTILE
            obase = p * _OUNIT

            def unit(i):
                ls = i >> 3
                sub = i & 7
                xv = x_v[pl.ds(xbase + ls * 128 + sub * _L, _L)]
                xv8 = xv * _D
                for o in range(_D):
                    val = plsc.load_gather(tab_v, [xv8 + o])
                    out_v[pl.ds(obase + ls * _OTILE + o * 128 + sub * _L,
                                _L)] = val

            plsc.parallel_loop(0, _D * _D, 1, unroll=8)(unit)

            start_x(u + _RING, p)
            for ls in range(_D):
                dst = ((lt * _D + ls) * _NBT + bt) * _OTILE
                pltpu.async_copy(
                    out_v.at[pl.ds(obase + ls * _OTILE, _OTILE)],
                    out_hbm.at[pl.ds(dst, _OTILE)], sem_out[p])
        return carry

    lax.fori_loop(0, _UNITS // _RING, k_body, 0)

    for slot in range(_RING):
        drain_out(slot)
        pltpu.make_async_copy(
            x_hbm.at[pl.ds(0, _XTILE)],
            x_v.at[pl.ds(slot * _XTILE, _XTILE)], sem_x[slot]).wait()


def _make_sc_call():
    mesh = plsc.VectorSubcoreMesh(core_axis_name="c", subcore_axis_name="s")
    return pl.kernel(
        _tiny_model_body,
        out_type=jax.ShapeDtypeStruct((_B * _SEQ * _D,), jnp.float32),
        mesh=mesh,
        compiler_params=pltpu.CompilerParams(needs_layout_passes=False),
        scratch_types=[
            pltpu.VMEM((_V * _D,), jnp.float32),
            pltpu.VMEM((_RING * _XTILE,), jnp.int32),
            pltpu.VMEM((_RING * _OUNIT,), jnp.float32),
            [pltpu.SemaphoreType.DMA] * _RING,
            [pltpu.SemaphoreType.DMA] * _RING,
        ],
    )


def kernel(x, embed_weight, proj_weight, proj_bias):
    b, l = x.shape
    x4 = (x.astype(jnp.int32).T
          .reshape(_NLT, _D, _NBT, 128)
          .transpose(0, 2, 1, 3)
          .reshape(-1))
    out_flat = _make_sc_call()(
        x4,
        embed_weight.reshape(-1).astype(jnp.float32),
        jnp.pad(proj_weight.reshape(-1).astype(jnp.float32), (0, 128 - _D * _D)),
        jnp.pad(proj_bias.astype(jnp.float32), (0, 128 - _D)),
    )
    return (out_flat.reshape(_SEQ, _NBT, _D, 128)
            .transpose(1, 3, 0, 2)
            .reshape(b, l, _D))


# speedup vs baseline: 1.1636x; 1.1636x over previous
import functools

import jax
import jax.numpy as jnp
from jax import lax
from jax.experimental import pallas as pl
from jax.experimental.pallas import tpu as pltpu
from jax.experimental.pallas import tpu_sc as plsc

_NC = 2
_NS = 16
_L = 16
_NW = _NC * _NS

_D = 8
_V = 16
_B = 16384
_SEQ = 200
_NLT = _SEQ // _D
_NBT = _B // 128
_BT_PER_W = _NBT // _NW
_UNITS = _BT_PER_W * _NLT
_RING = 5

_XTILE = _D * 128
_OTILE = _D * 128
_OUNIT = _D * _OTILE


def _tiny_model_body(x_hbm, emb_hbm, w_hbm, b_hbm, out_hbm,
                     tab_v, x_v, out_v, sem_x, sem_out):
    wid = lax.axis_index("s") * _NC + lax.axis_index("c")
    bt0 = wid * _BT_PER_W

    lanes = lax.iota(jnp.int32, _L)
    o_pat = lanes & 7
    half = lanes >> 3

    def with_weights(emb_v, w_v, b_v):
        pltpu.sync_copy(emb_hbm, emb_v)
        pltpu.sync_copy(w_hbm, w_v)
        pltpu.sync_copy(b_hbm, b_v)

        def build_tab(j, carry):
            k_pat = 2 * j + half
            acc = plsc.load_gather(b_v, [o_pat])
            for d in range(_D):
                ev = plsc.load_gather(emb_v, [k_pat * _D + d])
                wv = plsc.load_gather(w_v, [o_pat * _D + d])
                acc = acc + ev * wv
            tab_v[pl.ds(j * _L, _L)] = acc
            return carry

        lax.fori_loop(0, _V // 2, build_tab, 0)

    pl.run_scoped(
        with_weights,
        pltpu.VMEM((128,), jnp.float32),
        pltpu.VMEM((128,), jnp.float32),
        pltpu.VMEM((128,), jnp.float32),
    )

    def x_off(u):
        lt = lax.rem(u, _NLT)
        bt = bt0 + lax.div(u, _NLT)
        return (lt * _NBT + bt) * _XTILE

    def start_x(u, slot):
        return pltpu.async_copy(
            x_hbm.at[pl.ds(x_off(lax.min(u, _UNITS - 1)), _XTILE)],
            x_v.at[pl.ds(slot * _XTILE, _XTILE)], sem_x[slot])

    def wait_x(u, slot):
        pltpu.make_async_copy(
            x_hbm.at[pl.ds(x_off(lax.min(u, _UNITS - 1)), _XTILE)],
            x_v.at[pl.ds(slot * _XTILE, _XTILE)], sem_x[slot]).wait()

    def drain_out(slot):
        pltpu.make_async_copy(
            out_hbm.at[pl.ds(0, _OUNIT)],
            out_v.at[pl.ds(slot * _OUNIT, _OUNIT)], sem_out[slot]).wait()

    for slot in range(_RING):
        start_x(jnp.int32(slot), slot)

    def k_body(k, carry):
        for p in range(_RING):
            u = k * _RING + p
            lt = lax.rem(u, _NLT)
            bt = bt0 + lax.div(u, _NLT)
            wait_x(u, p)

            @pl.when(k >= 1)
            def _():
                drain_out(p)

            xbase = p * _XTILE
            obase = p * _OUNIT

            def unit(i):
                ls = i >> 3
                sub = i & 7
                xv = x_v[pl.ds(xbase + ls * 128 + sub * _L, _L)]
                xv8 = xv * _D
                for o in range(_D):
                    val = plsc.load_gather(tab_v, [xv8 + o])
                    out_v[pl.ds(obase + ls * _OTILE + o * 128 + sub * _L,
                                _L)] = val

            plsc.parallel_loop(0, _D * _D, 1, unroll=4)(unit)

            start_x(u + _RING, p)
            for ls in range(_D):
                dst = ((lt * _D + ls) * _NBT + bt) * _OTILE
                pltpu.async_copy(
                    out_v.at[pl.ds(obase + ls * _OTILE, _OTILE)],
                    out_hbm.at[pl.ds(dst, _OTILE)], sem_out[p])
        return carry

    lax.fori_loop(0, _UNITS // _RING, k_body, 0)

    for slot in range(_RING):
        drain_out(slot)
        pltpu.make_async_copy(
            x_hbm.at[pl.ds(0, _XTILE)],
            x_v.at[pl.ds(slot * _XTILE, _XTILE)], sem_x[slot]).wait()


def _make_sc_call():
    mesh = plsc.VectorSubcoreMesh(core_axis_name="c", subcore_axis_name="s")
    return pl.kernel(
        _tiny_model_body,
        out_type=jax.ShapeDtypeStruct((_B * _SEQ * _D,), jnp.float32),
        mesh=mesh,
        compiler_params=pltpu.CompilerParams(needs_layout_passes=False),
        scratch_types=[
            pltpu.VMEM((_V * _D,), jnp.float32),
            pltpu.VMEM((_RING * _XTILE,), jnp.int32),
            pltpu.VMEM((_RING * _OUNIT,), jnp.float32),
            [pltpu.SemaphoreType.DMA] * _RING,
            [pltpu.SemaphoreType.DMA] * _RING,
        ],
    )


def kernel(x, embed_weight, proj_weight, proj_bias):
    b, l = x.shape
    x4 = (x.astype(jnp.int32).T
          .reshape(_NLT, _D, _NBT, 128)
          .transpose(0, 2, 1, 3)
          .reshape(-1))
    out_flat = _make_sc_call()(
        x4,
        embed_weight.reshape(-1).astype(jnp.float32),
        jnp.pad(proj_weight.reshape(-1).astype(jnp.float32), (0, 128 - _D * _D)),
        jnp.pad(proj_bias.astype(jnp.float32), (0, 128 - _D)),
    )
    return (out_flat.reshape(_SEQ, _NBT, _D, 128)
            .transpose(1, 3, 0, 2)
            .reshape(b, l, _D))


# speedup vs baseline: 2.0577x; 1.7683x over previous
import functools

import jax
import jax.numpy as jnp
from jax import lax
from jax.experimental import pallas as pl
from jax.experimental.pallas import tpu as pltpu
from jax.experimental.pallas import tpu_sc as plsc

_NC = 2
_NS = 16
_L = 16
_NW = _NC * _NS

_D = 8
_V = 16
_B = 16384
_SEQ = 200
_NLT = _SEQ // _D
_NBT = _B // 128
_BT_PER_W = _NBT // _NW
_UNITS = _BT_PER_W * _NLT
_RING = 2
_UPS = 2

_XTILE = _D * 128
_OTILE = _D * 128
_OUNIT = _D * _OTILE


def _tiny_model_body(x_hbm, emb_hbm, w_hbm, b_hbm, out_hbm,
                     tab_v, x_v, out_v, sem_x, sem_out):
    wid = lax.axis_index("s") * _NC + lax.axis_index("c")
    bt0 = wid * _BT_PER_W

    lanes = lax.iota(jnp.int32, _L)
    o_pat = lanes & 7
    half = lanes >> 3

    def with_weights(emb_v, w_v, b_v):
        pltpu.sync_copy(emb_hbm, emb_v)
        pltpu.sync_copy(w_hbm, w_v)
        pltpu.sync_copy(b_hbm, b_v)

        def build_tab(j, carry):
            k_pat = 2 * j + half
            acc = plsc.load_gather(b_v, [o_pat])
            for d in range(_D):
                ev = plsc.load_gather(emb_v, [k_pat * _D + d])
                wv = plsc.load_gather(w_v, [o_pat * _D + d])
                acc = acc + ev * wv
            tab_v[pl.ds(j * _L, _L)] = acc
            return carry

        lax.fori_loop(0, _V // 2, build_tab, 0)

    pl.run_scoped(
        with_weights,
        pltpu.VMEM((128,), jnp.float32),
        pltpu.VMEM((128,), jnp.float32),
        pltpu.VMEM((128,), jnp.float32),
    )

    tcols = [plsc.load_gather(tab_v, [lanes * _D + o]) for o in range(_D)]
    _dnums = lax.GatherDimensionNumbers(
        offset_dims=(), collapsed_slice_dims=(0,), start_index_map=(0,))

    def _lookup(tcol, xv):
        return lax.gather(tcol, xv[:, None], dimension_numbers=_dnums,
                          slice_sizes=(1,),
                          mode=lax.GatherScatterMode.PROMISE_IN_BOUNDS)

    def x_off(u):
        lt = lax.rem(u, _NLT)
        bt = bt0 + lax.div(u, _NLT)
        return (lt * _NBT + bt) * _XTILE

    def start_x(u0, slot):
        for h in range(_UPS):
            pltpu.async_copy(
                x_hbm.at[pl.ds(x_off(lax.min(u0 + h, _UNITS - 1)), _XTILE)],
                x_v.at[pl.ds((slot * _UPS + h) * _XTILE, _XTILE)],
                sem_x[slot])

    def wait_x(slot):
        pltpu.make_async_copy(
            x_hbm.at[pl.ds(0, _UPS * _XTILE)],
            x_v.at[pl.ds(slot * _UPS * _XTILE, _UPS * _XTILE)],
            sem_x[slot]).wait()

    def drain_out(slot):
        pltpu.make_async_copy(
            out_hbm.at[pl.ds(0, _UPS * _OUNIT)],
            out_v.at[pl.ds(slot * _UPS * _OUNIT, _UPS * _OUNIT)],
            sem_out[slot]).wait()

    for slot in range(_RING):
        start_x(jnp.int32(slot * _UPS), slot)

    def k_body(k, carry):
        for p in range(_RING):
            u0 = (k * _RING + p) * _UPS
            wait_x(p)

            @pl.when(k >= 1)
            def _():
                drain_out(p)

            xbase = p * _UPS * _XTILE
            obase = p * _UPS * _OUNIT

            def unit(i):
                h = i >> 6
                ls = (i >> 3) & 7
                sub = i & 7
                xv = x_v[pl.ds(xbase + h * _XTILE + ls * 128 + sub * _L, _L)]
                for o in range(_D):
                    val = _lookup(tcols[o], xv)
                    out_v[pl.ds(obase + h * _OUNIT + ls * _OTILE + o * 128
                                + sub * _L, _L)] = val

            plsc.parallel_loop(0, _UPS * _D * _D, 1, unroll=4)(unit)

            start_x(u0 + _RING * _UPS, p)
            for h in range(_UPS):
                u = u0 + h
                lt = lax.rem(u, _NLT)
                bt = bt0 + lax.div(u, _NLT)
                for ls in range(_D):
                    dst = ((lt * _D + ls) * _NBT + bt) * _OTILE
                    pltpu.async_copy(
                        out_v.at[pl.ds(obase + h * _OUNIT + ls * _OTILE,
                                       _OTILE)],
                        out_hbm.at[pl.ds(dst, _OTILE)], sem_out[p])
        return carry

    lax.fori_loop(0, _UNITS // (_RING * _UPS), k_body, 0)

    for slot in range(_RING):
        drain_out(slot)
        wait_x(slot)


def _make_sc_call():
    mesh = plsc.VectorSubcoreMesh(core_axis_name="c", subcore_axis_name="s")
    return pl.kernel(
        _tiny_model_body,
        out_type=jax.ShapeDtypeStruct((_B * _SEQ * _D,), jnp.float32),
        mesh=mesh,
        compiler_params=pltpu.CompilerParams(needs_layout_passes=False),
        scratch_types=[
            pltpu.VMEM((_V * _D,), jnp.float32),
            pltpu.VMEM((_RING * _UPS * _XTILE,), jnp.int32),
            pltpu.VMEM((_RING * _UPS * _OUNIT,), jnp.float32),
            [pltpu.SemaphoreType.DMA] * _RING,
            [pltpu.SemaphoreType.DMA] * _RING,
        ],
    )


def kernel(x, embed_weight, proj_weight, proj_bias):
    b, l = x.shape
    x4 = (x.astype(jnp.int32).T
          .reshape(_NLT, _D, _NBT, 128)
          .transpose(0, 2, 1, 3)
          .reshape(-1))
    out_flat = _make_sc_call()(
        x4,
        embed_weight.reshape(-1).astype(jnp.float32),
        jnp.pad(proj_weight.reshape(-1).astype(jnp.float32), (0, 128 - _D * _D)),
        jnp.pad(proj_bias.astype(jnp.float32), (0, 128 - _D)),
    )
    return (out_flat.reshape(_SEQ, _NBT, _D, 128)
            .transpose(1, 3, 0, 2)
            .reshape(b, l, _D))


# speedup vs baseline: 2.0998x; 1.0205x over previous
import functools

import jax
import jax.numpy as jnp
from jax import lax
from jax.experimental import pallas as pl
from jax.experimental.pallas import tpu as pltpu
from jax.experimental.pallas import tpu_sc as plsc

_NC = 2
_NS = 16
_L = 16
_NW = _NC * _NS

_D = 8
_V = 16
_B = 16384
_SEQ = 200
_NLT = _SEQ // _D
_NBT = _B // 128
_BT_PER_W = _NBT // _NW
_UNITS = _BT_PER_W * _NLT
_RING = 2
_UPS = 5

_XTILE = _D * 128
_OTILE = _D * 128
_OUNIT = _D * _OTILE


def _tiny_model_body(x_hbm, emb_hbm, w_hbm, b_hbm, out_hbm,
                     tab_v, x_v, out_v, sem_x, sem_out):
    wid = lax.axis_index("s") * _NC + lax.axis_index("c")
    bt0 = wid * _BT_PER_W

    lanes = lax.iota(jnp.int32, _L)
    o_pat = lanes & 7
    half = lanes >> 3

    def with_weights(emb_v, w_v, b_v):
        pltpu.sync_copy(emb_hbm, emb_v)
        pltpu.sync_copy(w_hbm, w_v)
        pltpu.sync_copy(b_hbm, b_v)

        def build_tab(j, carry):
            k_pat = 2 * j + half
            acc = plsc.load_gather(b_v, [o_pat])
            for d in range(_D):
                ev = plsc.load_gather(emb_v, [k_pat * _D + d])
                wv = plsc.load_gather(w_v, [o_pat * _D + d])
                acc = acc + ev * wv
            tab_v[pl.ds(j * _L, _L)] = acc
            return carry

        lax.fori_loop(0, _V // 2, build_tab, 0)

    pl.run_scoped(
        with_weights,
        pltpu.VMEM((128,), jnp.float32),
        pltpu.VMEM((128,), jnp.float32),
        pltpu.VMEM((128,), jnp.float32),
    )

    tcols = [plsc.load_gather(tab_v, [lanes * _D + o]) for o in range(_D)]
    _dnums = lax.GatherDimensionNumbers(
        offset_dims=(), collapsed_slice_dims=(0,), start_index_map=(0,))

    def _lookup(tcol, xv):
        return lax.gather(tcol, xv[:, None], dimension_numbers=_dnums,
                          slice_sizes=(1,),
                          mode=lax.GatherScatterMode.PROMISE_IN_BOUNDS)

    def x_off(u):
        lt = lax.rem(u, _NLT)
        bt = bt0 + lax.div(u, _NLT)
        return (lt * _NBT + bt) * _XTILE

    def start_x(u0, slot):
        for h in range(_UPS):
            pltpu.async_copy(
                x_hbm.at[pl.ds(x_off(lax.min(u0 + h, _UNITS - 1)), _XTILE)],
                x_v.at[pl.ds((slot * _UPS + h) * _XTILE, _XTILE)],
                sem_x[slot])

    def wait_x(slot):
        pltpu.make_async_copy(
            x_hbm.at[pl.ds(0, _UPS * _XTILE)],
            x_v.at[pl.ds(slot * _UPS * _XTILE, _UPS * _XTILE)],
            sem_x[slot]).wait()

    def drain_out(slot):
        pltpu.make_async_copy(
            out_hbm.at[pl.ds(0, _UPS * _OUNIT)],
            out_v.at[pl.ds(slot * _UPS * _OUNIT, _UPS * _OUNIT)],
            sem_out[slot]).wait()

    for slot in range(_RING):
        start_x(jnp.int32(slot * _UPS), slot)

    def k_body(k, carry):
        for p in range(_RING):
            u0 = (k * _RING + p) * _UPS
            wait_x(p)

            @pl.when(k >= 1)
            def _():
                drain_out(p)

            xbase = p * _UPS * _XTILE
            obase = p * _UPS * _OUNIT

            def unit(i):
                h = i >> 6
                ls = (i >> 3) & 7
                sub = i & 7
                xv = x_v[pl.ds(xbase + h * _XTILE + ls * 128 + sub * _L, _L)]
                for o in range(_D):
                    val = _lookup(tcols[o], xv)
                    out_v[pl.ds(obase + h * _OUNIT + ls * _OTILE + o * 128
                                + sub * _L, _L)] = val

            plsc.parallel_loop(0, _UPS * _D * _D, 1, unroll=4)(unit)

            start_x(u0 + _RING * _UPS, p)
            for h in range(_UPS):
                u = u0 + h
                lt = lax.rem(u, _NLT)
                bt = bt0 + lax.div(u, _NLT)
                for ls in range(_D):
                    dst = ((lt * _D + ls) * _NBT + bt) * _OTILE
                    pltpu.async_copy(
                        out_v.at[pl.ds(obase + h * _OUNIT + ls * _OTILE,
                                       _OTILE)],
                        out_hbm.at[pl.ds(dst, _OTILE)], sem_out[p])
        return carry

    lax.fori_loop(0, _UNITS // (_RING * _UPS), k_body, 0)

    for slot in range(_RING):
        drain_out(slot)
        wait_x(slot)


def _make_sc_call():
    mesh = plsc.VectorSubcoreMesh(core_axis_name="c", subcore_axis_name="s")
    return pl.kernel(
        _tiny_model_body,
        out_type=jax.ShapeDtypeStruct((_B * _SEQ * _D,), jnp.float32),
        mesh=mesh,
        compiler_params=pltpu.CompilerParams(needs_layout_passes=False),
        scratch_types=[
            pltpu.VMEM((_V * _D,), jnp.float32),
            pltpu.VMEM((_RING * _UPS * _XTILE,), jnp.int32),
            pltpu.VMEM((_RING * _UPS * _OUNIT,), jnp.float32),
            [pltpu.SemaphoreType.DMA] * _RING,
            [pltpu.SemaphoreType.DMA] * _RING,
        ],
    )


def kernel(x, embed_weight, proj_weight, proj_bias):
    b, l = x.shape
    x4 = (x.astype(jnp.int32).T
          .reshape(_NLT, _D, _NBT, 128)
          .transpose(0, 2, 1, 3)
          .reshape(-1))
    out_flat = _make_sc_call()(
        x4,
        embed_weight.reshape(-1).astype(jnp.float32),
        jnp.pad(proj_weight.reshape(-1).astype(jnp.float32), (0, 128 - _D * _D)),
        jnp.pad(proj_bias.astype(jnp.float32), (0, 128 - _D)),
    )
    return (out_flat.reshape(_SEQ, _NBT, _D, 128)
            .transpose(1, 3, 0, 2)
            .reshape(b, l, _D))


# speedup vs baseline: 2.1059x; 1.0029x over previous
import functools

import jax
import jax.numpy as jnp
from jax import lax
from jax.experimental import pallas as pl
from jax.experimental.pallas import tpu as pltpu
from jax.experimental.pallas import tpu_sc as plsc

_NC = 2
_NS = 16
_L = 16
_NW = _NC * _NS

_D = 8
_V = 16
_B = 16384
_SEQ = 200
_NLT = _SEQ // _D
_NBT = _B // 128
_BT_PER_W = _NBT // _NW
_UNITS = _BT_PER_W * _NLT
_RING = 2
_UPS = 5

_XTILE = _D * 128
_OTILE = _D * 128
_OUNIT = _D * _OTILE


def _tiny_model_body(x_hbm, emb_hbm, w_hbm, b_hbm, out_hbm,
                     tab_v, x_v, out_v, sem_x, sem_out):
    wid = lax.axis_index("s") * _NC + lax.axis_index("c")
    bt0 = wid * _BT_PER_W

    lanes = lax.iota(jnp.int32, _L)
    o_pat = lanes & 7
    half = lanes >> 3

    def with_weights(emb_v, w_v, b_v):
        pltpu.sync_copy(emb_hbm, emb_v)
        pltpu.sync_copy(w_hbm, w_v)
        pltpu.sync_copy(b_hbm, b_v)

        def build_tab(j, carry):
            k_pat = 2 * j + half
            acc = plsc.load_gather(b_v, [o_pat])
            for d in range(_D):
                ev = plsc.load_gather(emb_v, [k_pat * _D + d])
                wv = plsc.load_gather(w_v, [o_pat * _D + d])
                acc = acc + ev * wv
            tab_v[pl.ds(j * _L, _L)] = acc
            return carry

        lax.fori_loop(0, _V // 2, build_tab, 0)

    pl.run_scoped(
        with_weights,
        pltpu.VMEM((128,), jnp.float32),
        pltpu.VMEM((128,), jnp.float32),
        pltpu.VMEM((128,), jnp.float32),
    )

    tcols = [plsc.load_gather(tab_v, [lanes * _D + o]) for o in range(_D)]
    _dnums = lax.GatherDimensionNumbers(
        offset_dims=(), collapsed_slice_dims=(0,), start_index_map=(0,))

    def _lookup(tcol, xv):
        return lax.gather(tcol, xv[:, None], dimension_numbers=_dnums,
                          slice_sizes=(1,),
                          mode=lax.GatherScatterMode.PROMISE_IN_BOUNDS)

    def x_off(u):
        lt = lax.rem(u, _NLT)
        bt = bt0 + lax.div(u, _NLT)
        return (lt * _NBT + bt) * _XTILE

    def start_x(u0, slot):
        for h in range(_UPS):
            pltpu.async_copy(
                x_hbm.at[pl.ds(x_off(lax.min(u0 + h, _UNITS - 1)), _XTILE)],
                x_v.at[pl.ds((slot * _UPS + h) * _XTILE, _XTILE)],
                sem_x[slot])

    def wait_x(slot):
        pltpu.make_async_copy(
            x_hbm.at[pl.ds(0, _UPS * _XTILE)],
            x_v.at[pl.ds(slot * _UPS * _XTILE, _UPS * _XTILE)],
            sem_x[slot]).wait()

    def drain_out(slot):
        pltpu.make_async_copy(
            out_hbm.at[pl.ds(0, _UPS * _OUNIT)],
            out_v.at[pl.ds(slot * _UPS * _OUNIT, _UPS * _OUNIT)],
            sem_out[slot]).wait()

    for slot in range(_RING):
        start_x(jnp.int32(slot * _UPS), slot)

    def k_body(k, carry):
        for p in range(_RING):
            u0 = (k * _RING + p) * _UPS
            wait_x(p)

            @pl.when(k >= 1)
            def _():
                drain_out(p)

            xbase = p * _UPS * _XTILE
            obase = p * _UPS * _OUNIT

            def unit(i):
                h = i >> 6
                ls = (i >> 3) & 7
                sub = i & 7
                xv = x_v[pl.ds(xbase + h * _XTILE + ls * 128 + sub * _L, _L)]
                for o in range(_D):
                    val = _lookup(tcols[o], xv)
                    out_v[pl.ds(obase + h * _OUNIT + ls * _OTILE + o * 128
                                + sub * _L, _L)] = val

            plsc.parallel_loop(0, _UPS * _D * _D, 1, unroll=8)(unit)

            start_x(u0 + _RING * _UPS, p)
            for h in range(_UPS):
                u = u0 + h
                lt = lax.rem(u, _NLT)
                bt = bt0 + lax.div(u, _NLT)
                for ls in range(_D):
                    dst = ((lt * _D + ls) * _NBT + bt) * _OTILE
                    pltpu.async_copy(
                        out_v.at[pl.ds(obase + h * _OUNIT + ls * _OTILE,
                                       _OTILE)],
                        out_hbm.at[pl.ds(dst, _OTILE)], sem_out[p])
        return carry

    lax.fori_loop(0, _UNITS // (_RING * _UPS), k_body, 0)

    for slot in range(_RING):
        drain_out(slot)
        wait_x(slot)


def _make_sc_call():
    mesh = plsc.VectorSubcoreMesh(core_axis_name="c", subcore_axis_name="s")
    return pl.kernel(
        _tiny_model_body,
        out_type=jax.ShapeDtypeStruct((_B * _SEQ * _D,), jnp.float32),
        mesh=mesh,
        compiler_params=pltpu.CompilerParams(needs_layout_passes=False),
        scratch_types=[
            pltpu.VMEM((_V * _D,), jnp.float32),
            pltpu.VMEM((_RING * _UPS * _XTILE,), jnp.int32),
            pltpu.VMEM((_RING * _UPS * _OUNIT,), jnp.float32),
            [pltpu.SemaphoreType.DMA] * _RING,
            [pltpu.SemaphoreType.DMA] * _RING,
        ],
    )


def kernel(x, embed_weight, proj_weight, proj_bias):
    b, l = x.shape
    x4 = (x.astype(jnp.int32).T
          .reshape(_NLT, _D, _NBT, 128)
          .transpose(0, 2, 1, 3)
          .reshape(-1))
    out_flat = _make_sc_call()(
        x4,
        embed_weight.reshape(-1).astype(jnp.float32),
        jnp.pad(proj_weight.reshape(-1).astype(jnp.float32), (0, 128 - _D * _D)),
        jnp.pad(proj_bias.astype(jnp.float32), (0, 128 - _D)),
    )
    return (out_flat.reshape(_SEQ, _NBT, _D, 128)
            .transpose(1, 3, 0, 2)
            .reshape(b, l, _D))
